# baseline (speedup 1.0000x reference)
import functools

import jax
import jax.numpy as jnp
from jax import lax
from jax.experimental import pallas as pl
from jax.experimental.pallas import tpu as pltpu
from jax.experimental.pallas import tpu_sc as plsc

_B = 16384
_C = 784
_NB = 64
_NC = 208
_NN = 512
_KS = 16
_NOUT = _NB + _NC + _NN // _KS
_L = 16
_NCORES = 2
_NSUB = 16
_NW = _NCORES * _NSUB
_ROWS_PER_W = _B // _NW
_GROUPS = _ROWS_PER_W // _L
_VPC = _C // _L
_BSC = 4096


def _tc_row_slice(x2d, nrows):
    rb = 1024

    def body(x_ref, o_ref):
        o_ref[...] = x_ref[...]

    return pl.pallas_call(
        body,
        out_shape=jax.ShapeDtypeStruct((nrows, _C), jnp.float32),
        grid=(nrows // rb,),
        in_specs=[pl.BlockSpec((rb, _C), lambda i: (i, 0))],
        out_specs=pl.BlockSpec((rb, _C), lambda i: (i, 0)),
    )(x2d)


def _sc_partial_rowsum(x2d, a, cvec):
    bs = x2d.shape[0]
    rows_per_w = bs // _NW
    groups = rows_per_w // _L
    mesh = plsc.VectorSubcoreMesh(core_axis_name="c", subcore_axis_name="s")

    @functools.partial(
        pl.kernel,
        mesh=mesh,
        out_type=jax.ShapeDtypeStruct((bs * _L,), jnp.float32),
        scratch_types=[
            pltpu.VMEM((_L, _C), jnp.float32),
            pltpu.VMEM((_L, _C), jnp.float32),
            pltpu.VMEM((_L * _L,), jnp.float32),
            pltpu.VMEM((_L * _L,), jnp.float32),
            pltpu.VMEM((_C,), jnp.float32),
            pltpu.VMEM((_L,), jnp.float32),
            pltpu.SemaphoreType.DMA,
            pltpu.SemaphoreType.DMA,
            pltpu.SemaphoreType.DMA,
            pltpu.SemaphoreType.DMA,
        ],
    )
    def k(x_hbm, a_hbm, c_hbm, p_hbm, buf0, buf1, pb0, pb1, a_v, c_v,
          isem0, isem1, osem0, osem1):
        wid = lax.axis_index("s") * _NCORES + lax.axis_index("c")
        base = wid * rows_per_w
        pltpu.sync_copy(a_hbm, a_v)
        pltpu.sync_copy(c_hbm, c_v)
        cv = c_v[...]

        bufs = (buf0, buf1)
        pbs = (pb0, pb1)
        isems = (isem0, isem1)
        osems = (osem0, osem1)

        def in_slice(g):
            row0 = base + g * _L
            return x_hbm.at[pl.ds(row0, _L), :]

        def out_slice(g):
            row0 = base + g * _L
            return p_hbm.at[pl.ds(row0 * _L, _L * _L)]

        pltpu.async_copy(in_slice(0), buf0, isem0)

        def step(i, carry):
            for s in range(2):
                g = 2 * i + s
                buf, pb = bufs[s], pbs[s]
                isem, osem = isems[s], osems[s]
                o = 1 - s

                @pl.when(g + 1 < groups)
                def _():
                    pltpu.async_copy(in_slice(g + 1), bufs[o], isems[o])

                pltpu.make_async_copy(in_slice(g), buf, isem).wait()

                @pl.when(i > 0)
                def _():
                    pltpu.make_async_copy(pb, out_slice(g), osem).wait()

                for half in range(2):
                    @plsc.parallel_loop(0, _VPC, 1, unroll=7,
                                        carry=(cv,) * (_L // 2))
                    def jloop(j, accs, half=half):
                        off = j * _L
                        aj = a_v[pl.ds(off, _L)]
                        return tuple(
                            accs[r]
                            + buf[half * (_L // 2) + r, pl.ds(off, _L)] * aj
                            for r in range(_L // 2))
                    accs = jloop
                    for r in range(_L // 2):
                        rr = half * (_L // 2) + r
                        pb[pl.ds(rr * _L, _L)] = accs[r]
                pltpu.async_copy(pb, out_slice(g), osem)
            return carry

        lax.fori_loop(0, groups // 2, step, 0)
        pltpu.make_async_copy(pb0, out_slice(groups - 2), osem0).wait()
        pltpu.make_async_copy(pb1, out_slice(groups - 1), osem1).wait()

    return k(x2d, a, cvec)


def _tc_final_rowsum(p, moh):
    prows = p.shape[0]

    def body(p_ref, m_ref, o_ref):
        o_ref[...] = jax.lax.dot(
            p_ref[...], m_ref[...],
            precision=jax.lax.Precision.HIGHEST,
            preferred_element_type=jnp.float32)

    return pl.pallas_call(
        body,
        out_shape=jax.ShapeDtypeStruct((prows, 8), jnp.float32),
        grid=(2,),
        in_specs=[
            pl.BlockSpec((prows // 2, 128), lambda i: (i, 0)),
            pl.BlockSpec((128, 8), lambda i: (0, 0)),
        ],
        out_specs=pl.BlockSpec((prows // 2, 8), lambda i: (i, 0)),
    )(p, moh)


def _tc_main_rowsum(x2d, a2d, c11, row0, nrows):
    rb = 1024
    nblk = nrows // rb
    blk0 = row0 // rb

    def body(x_ref, a_ref, c_ref, o_ref):
        o_ref[...] = jnp.sum(x_ref[...] * a_ref[...], axis=1,
                             keepdims=True) + c_ref[...]

    return pl.pallas_call(
        body,
        out_shape=jax.ShapeDtypeStruct((nrows, 1), jnp.float32),
        grid=(nblk,),
        in_specs=[
            pl.BlockSpec((rb, _C), lambda i: (blk0 + i, 0)),
            pl.BlockSpec((1, _C), lambda i: (0, 0)),
            pl.BlockSpec((1, 1), lambda i: (0, 0)),
        ],
        out_specs=pl.BlockSpec((rb, 1), lambda i: (i, 0)),
    )(x2d, a2d, c11)


def kernel(input_linear, w1, b1, w2, b2, K, train_size, num_cat_variable,
           num_num_variable, num_bin_variable,
           vectorized_cate_col_name_num_list):
    x2d = input_linear.reshape(_B, _C)

    counts = jnp.asarray(vectorized_cate_col_name_num_list, dtype=jnp.int32)
    cum = jnp.cumsum(counts)
    jj = jnp.arange(_NC, dtype=jnp.int32)
    seg = jnp.sum((cum[None, :] <= jj[:, None]).astype(jnp.int32), axis=1)
    seg = jnp.minimum(seg, _NC - 1)
    gmap = jnp.concatenate([
        jnp.arange(_NB, dtype=jnp.int32),
        _NB + seg,
        _NB + _NC + jnp.arange(_NN, dtype=jnp.int32) // _KS,
    ])
    onehot = (gmap[:, None] == jnp.arange(_NOUT, dtype=jnp.int32)[None, :])
    w2g = jnp.dot(onehot.astype(jnp.float32), w2,
                  precision=jax.lax.Precision.HIGHEST)
    a = w1 * w2g
    cconst = jnp.dot(b1, w2g, precision=jax.lax.Precision.HIGHEST) + jnp.sum(b2)
    cvec = jnp.full((_L,), cconst / _L, dtype=jnp.float32)

    lane = jnp.arange(128, dtype=jnp.int32)
    moh = (lane[:, None] // _L == jnp.arange(8, dtype=jnp.int32)[None, :])
    moh = moh.astype(jnp.float32)

    x_sc = _tc_row_slice(x2d, _BSC)
    p = _sc_partial_rowsum(x_sc, a, cvec)
    out_tc = _tc_main_rowsum(x2d, a.reshape(1, _C),
                             jnp.full((1, 1), cconst, jnp.float32),
                             _BSC, _B - _BSC)
    out_sc = _tc_final_rowsum(p.reshape(_BSC * _L // 128, 128), moh)
    return jnp.concatenate([out_sc.reshape(_BSC, 1), out_tc], axis=0)


# speedup vs baseline: 1.0513x; 1.0513x over previous
import functools

import jax
import jax.numpy as jnp
from jax import lax
from jax.experimental import pallas as pl
from jax.experimental.pallas import tpu as pltpu
from jax.experimental.pallas import tpu_sc as plsc

_B = 16384
_C = 784
_NB = 64
_NC = 208
_NN = 512
_KS = 16
_NOUT = _NB + _NC + _NN // _KS
_L = 16
_NCORES = 2
_NSUB = 16
_NW = _NCORES * _NSUB
_ROWS_PER_W = _B // _NW
_GROUPS = _ROWS_PER_W // _L
_VPC = _C // _L
_BSC = 5120


def _sc_partial_rowsum(x2d, a, cvec, bs):
    rows_per_w = bs // _NW
    groups = rows_per_w // _L
    mesh = plsc.VectorSubcoreMesh(core_axis_name="c", subcore_axis_name="s")

    @functools.partial(
        pl.kernel,
        mesh=mesh,
        out_type=jax.ShapeDtypeStruct((bs * _L,), jnp.float32),
        scratch_types=[
            pltpu.VMEM((_L, _C), jnp.float32),
            pltpu.VMEM((_L, _C), jnp.float32),
            pltpu.VMEM((_L * _L,), jnp.float32),
            pltpu.VMEM((_L * _L,), jnp.float32),
            pltpu.VMEM((_C,), jnp.float32),
            pltpu.VMEM((_L,), jnp.float32),
            pltpu.SemaphoreType.DMA,
            pltpu.SemaphoreType.DMA,
            pltpu.SemaphoreType.DMA,
            pltpu.SemaphoreType.DMA,
        ],
    )
    def k(x_hbm, a_hbm, c_hbm, p_hbm, buf0, buf1, pb0, pb1, a_v, c_v,
          isem0, isem1, osem0, osem1):
        wid = lax.axis_index("s") * _NCORES + lax.axis_index("c")
        base = wid * rows_per_w
        pltpu.sync_copy(a_hbm, a_v)
        pltpu.sync_copy(c_hbm, c_v)
        cv = c_v[...]

        bufs = (buf0, buf1)
        pbs = (pb0, pb1)
        isems = (isem0, isem1)
        osems = (osem0, osem1)

        def in_slice(g):
            row0 = base + g * _L
            return x_hbm.at[pl.ds(row0, _L), :]

        def out_slice(g):
            row0 = base + g * _L
            return p_hbm.at[pl.ds(row0 * _L, _L * _L)]

        pltpu.async_copy(in_slice(0), buf0, isem0)

        def step(i, carry):
            for s in range(2):
                g = 2 * i + s
                buf, pb = bufs[s], pbs[s]
                isem, osem = isems[s], osems[s]
                o = 1 - s

                @pl.when(g + 1 < groups)
                def _():
                    pltpu.async_copy(in_slice(g + 1), bufs[o], isems[o])

                pltpu.make_async_copy(in_slice(g), buf, isem).wait()

                @pl.when(i > 0)
                def _():
                    pltpu.make_async_copy(pb, out_slice(g), osem).wait()

                for half in range(2):
                    @plsc.parallel_loop(0, _VPC, 1, unroll=7,
                                        carry=(cv,) * (_L // 2))
                    def jloop(j, accs, half=half):
                        off = j * _L
                        aj = a_v[pl.ds(off, _L)]
                        return tuple(
                            accs[r]
                            + buf[half * (_L // 2) + r, pl.ds(off, _L)] * aj
                            for r in range(_L // 2))
                    accs = jloop
                    for r in range(_L // 2):
                        rr = half * (_L // 2) + r
                        pb[pl.ds(rr * _L, _L)] = accs[r]
                pltpu.async_copy(pb, out_slice(g), osem)
            return carry

        lax.fori_loop(0, groups // 2, step, 0)
        pltpu.make_async_copy(pb0, out_slice(groups - 2), osem0).wait()
        pltpu.make_async_copy(pb1, out_slice(groups - 1), osem1).wait()

    return k(x2d, a, cvec)


def _tc_final_rowsum(p, out_full, bs):
    rb = 1024

    def body(p_ref, of_ref, o_ref):
        del of_ref
        o_ref[...] = jnp.sum(p_ref[...], axis=1, keepdims=True)

    return pl.pallas_call(
        body,
        out_shape=jax.ShapeDtypeStruct((_B, 1), jnp.float32),
        grid=(bs // rb,),
        in_specs=[
            pl.BlockSpec((rb, _L), lambda i: (i, 0)),
            pl.BlockSpec((rb, 1), lambda i: (i, 0)),
        ],
        out_specs=pl.BlockSpec((rb, 1), lambda i: (i, 0)),
        input_output_aliases={1: 0},
    )(p, out_full)


def _tc_main_rowsum(x2d, a2d, c11, row0, nrows):
    rb = 1024
    nblk = nrows // rb
    blk0 = row0 // rb

    def body(x_ref, a_ref, c_ref, o_ref):
        o_ref[...] = jnp.sum(x_ref[...] * a_ref[...], axis=1,
                             keepdims=True) + c_ref[...]

    return pl.pallas_call(
        body,
        out_shape=jax.ShapeDtypeStruct((_B, 1), jnp.float32),
        grid=(nblk,),
        in_specs=[
            pl.BlockSpec((rb, _C), lambda i: (blk0 + i, 0)),
            pl.BlockSpec((1, _C), lambda i: (0, 0)),
            pl.BlockSpec((1, 1), lambda i: (0, 0)),
        ],
        out_specs=pl.BlockSpec((rb, 1), lambda i: (blk0 + i, 0)),
    )(x2d, a2d, c11)


def kernel(input_linear, w1, b1, w2, b2, K, train_size, num_cat_variable,
           num_num_variable, num_bin_variable,
           vectorized_cate_col_name_num_list):
    x2d = input_linear.reshape(_B, _C)

    counts = jnp.asarray(vectorized_cate_col_name_num_list, dtype=jnp.int32)
    cum = jnp.cumsum(counts)
    jj = jnp.arange(_NC, dtype=jnp.int32)
    seg = jnp.sum((cum[None, :] <= jj[:, None]).astype(jnp.int32), axis=1)
    seg = jnp.minimum(seg, _NC - 1)
    gmap = jnp.concatenate([
        jnp.arange(_NB, dtype=jnp.int32),
        _NB + seg,
        _NB + _NC + jnp.arange(_NN, dtype=jnp.int32) // _KS,
    ])
    onehot = (gmap[:, None] == jnp.arange(_NOUT, dtype=jnp.int32)[None, :])
    w2g = jnp.dot(onehot.astype(jnp.float32), w2,
                  precision=jax.lax.Precision.HIGHEST)
    a = w1 * w2g
    cconst = jnp.dot(b1, w2g, precision=jax.lax.Precision.HIGHEST) + jnp.sum(b2)
    cvec = jnp.full((_L,), cconst / _L, dtype=jnp.float32)

    p = _sc_partial_rowsum(x2d, a, cvec, _BSC)
    out_full = _tc_main_rowsum(x2d, a.reshape(1, _C),
                               jnp.full((1, 1), cconst, jnp.float32),
                               _BSC, _B - _BSC)
    return _tc_final_rowsum(p.reshape(_BSC, _L), out_full, _BSC)


# speedup vs baseline: 1.1267x; 1.0717x over previous
import functools

import jax
import jax.numpy as jnp
from jax import lax
from jax.experimental import pallas as pl
from jax.experimental.pallas import tpu as pltpu
from jax.experimental.pallas import tpu_sc as plsc

_B = 16384
_C = 784
_NB = 64
_NC = 208
_NN = 512
_KS = 16
_NOUT = _NB + _NC + _NN // _KS
_L = 16
_NCORES = 2
_NSUB = 16
_NW = _NCORES * _NSUB
_ROWS_PER_W = _B // _NW
_GROUPS = _ROWS_PER_W // _L
_VPC = _C // _L
_BSC = 5120


def _sc_partial_rowsum(x2d, a, cvec, bs):
    rows_per_w = bs // _NW
    groups = rows_per_w // _L
    mesh = plsc.VectorSubcoreMesh(core_axis_name="c", subcore_axis_name="s")

    @functools.partial(
        pl.kernel,
        mesh=mesh,
        out_type=jax.ShapeDtypeStruct((bs * _L,), jnp.float32),
        scratch_types=[
            pltpu.VMEM((_L, _C), jnp.float32),
            pltpu.VMEM((_L, _C), jnp.float32),
            pltpu.VMEM((_L * _L,), jnp.float32),
            pltpu.VMEM((_L * _L,), jnp.float32),
            pltpu.VMEM((_C,), jnp.float32),
            pltpu.VMEM((_L,), jnp.float32),
            pltpu.SemaphoreType.DMA,
            pltpu.SemaphoreType.DMA,
            pltpu.SemaphoreType.DMA,
            pltpu.SemaphoreType.DMA,
        ],
    )
    def k(x_hbm, a_hbm, c_hbm, p_hbm, buf0, buf1, pb0, pb1, a_v, c_v,
          isem0, isem1, osem0, osem1):
        wid = lax.axis_index("s") * _NCORES + lax.axis_index("c")
        base = wid * rows_per_w
        pltpu.sync_copy(a_hbm, a_v)
        pltpu.sync_copy(c_hbm, c_v)
        cv = c_v[...]

        bufs = (buf0, buf1)
        pbs = (pb0, pb1)
        isems = (isem0, isem1)
        osems = (osem0, osem1)

        def in_slice(g):
            row0 = base + g * _L
            return x_hbm.at[pl.ds(row0, _L), :]

        def out_slice(g):
            row0 = base + g * _L
            return p_hbm.at[pl.ds(row0 * _L, _L * _L)]

        pltpu.async_copy(in_slice(0), buf0, isem0)

        def step(i, carry):
            for s in range(2):
                g = 2 * i + s
                buf, pb = bufs[s], pbs[s]
                isem, osem = isems[s], osems[s]
                o = 1 - s

                @pl.when(g + 1 < groups)
                def _():
                    pltpu.async_copy(in_slice(g + 1), bufs[o], isems[o])

                pltpu.make_async_copy(in_slice(g), buf, isem).wait()

                @pl.when(i > 0)
                def _():
                    pltpu.make_async_copy(pb, out_slice(g), osem).wait()

                for half in range(2):
                    @plsc.parallel_loop(0, _VPC, 1, unroll=7,
                                        carry=(cv,) * (_L // 2))
                    def jloop(j, accs, half=half):
                        off = j * _L
                        aj = a_v[pl.ds(off, _L)]
                        return tuple(
                            accs[r]
                            + buf[half * (_L // 2) + r, pl.ds(off, _L)] * aj
                            for r in range(_L // 2))
                    accs = jloop
                    for r in range(_L // 2):
                        rr = half * (_L // 2) + r
                        pb[pl.ds(rr * _L, _L)] = accs[r]
                pltpu.async_copy(pb, out_slice(g), osem)
            return carry

        lax.fori_loop(0, groups // 2, step, 0)
        pltpu.make_async_copy(pb0, out_slice(groups - 2), osem0).wait()
        pltpu.make_async_copy(pb1, out_slice(groups - 1), osem1).wait()

    return k(x2d, a, cvec)


def _tc_final_rowsum(p, moh, bs):
    prows = bs * _L // 128

    def body(p_ref, m_ref, o_ref):
        o_ref[...] = jax.lax.dot(
            p_ref[...], m_ref[...],
            precision=jax.lax.Precision.HIGHEST,
            preferred_element_type=jnp.float32)

    return pl.pallas_call(
        body,
        out_shape=jax.ShapeDtypeStruct((prows, 8), jnp.float32),
        grid=(2,),
        in_specs=[
            pl.BlockSpec((prows // 2, 128), lambda i: (i, 0)),
            pl.BlockSpec((128, 8), lambda i: (0, 0)),
        ],
        out_specs=pl.BlockSpec((prows // 2, 8), lambda i: (i, 0)),
    )(p, moh)


def _tc_main_rowsum(x2d, a2d, c11, row0, nrows):
    rb = 1024
    nblk = nrows // rb
    blk0 = row0 // rb

    def body(x_ref, a_ref, c_ref, o_ref):
        o_ref[...] = jnp.sum(x_ref[...] * a_ref[...], axis=1,
                             keepdims=True) + c_ref[...]

    return pl.pallas_call(
        body,
        out_shape=jax.ShapeDtypeStruct((nrows, 1), jnp.float32),
        grid=(nblk,),
        in_specs=[
            pl.BlockSpec((rb, _C), lambda i: (blk0 + i, 0)),
            pl.BlockSpec((1, _C), lambda i: (0, 0)),
            pl.BlockSpec((1, 1), lambda i: (0, 0)),
        ],
        out_specs=pl.BlockSpec((rb, 1), lambda i: (i, 0)),
    )(x2d, a2d, c11)


def kernel(input_linear, w1, b1, w2, b2, K, train_size, num_cat_variable,
           num_num_variable, num_bin_variable,
           vectorized_cate_col_name_num_list):
    x2d = input_linear.reshape(_B, _C)

    counts = jnp.asarray(vectorized_cate_col_name_num_list, dtype=jnp.int32)
    cum = jnp.cumsum(counts)
    jj = jnp.arange(_NC, dtype=jnp.int32)
    seg = jnp.sum((cum[None, :] <= jj[:, None]).astype(jnp.int32), axis=1)
    seg = jnp.minimum(seg, _NC - 1)
    gmap = jnp.concatenate([
        jnp.arange(_NB, dtype=jnp.int32),
        _NB + seg,
        _NB + _NC + jnp.arange(_NN, dtype=jnp.int32) // _KS,
    ])
    onehot = (gmap[:, None] == jnp.arange(_NOUT, dtype=jnp.int32)[None, :])
    w2g = jnp.dot(onehot.astype(jnp.float32), w2,
                  precision=jax.lax.Precision.HIGHEST)
    a = w1 * w2g
    cconst = jnp.dot(b1, w2g, precision=jax.lax.Precision.HIGHEST) + jnp.sum(b2)
    cvec = jnp.full((_L,), cconst / _L, dtype=jnp.float32)

    lane = jnp.arange(128, dtype=jnp.int32)
    moh = (lane[:, None] // _L == jnp.arange(8, dtype=jnp.int32)[None, :])
    moh = moh.astype(jnp.float32)

    p = _sc_partial_rowsum(x2d, a, cvec, _BSC)
    out_tc = _tc_main_rowsum(x2d, a.reshape(1, _C),
                             jnp.full((1, 1), cconst, jnp.float32),
                             _BSC, _B - _BSC)
    out_sc = _tc_final_rowsum(p.reshape(_BSC * _L // 128, 128), moh, _BSC)
    return jnp.concatenate([out_sc.reshape(_BSC, 1), out_tc], axis=0)


# speedup vs baseline: 1.1276x; 1.0008x over previous
import functools

import jax
import jax.numpy as jnp
from jax import lax
from jax.experimental import pallas as pl
from jax.experimental.pallas import tpu as pltpu
from jax.experimental.pallas import tpu_sc as plsc

_B = 16384
_C = 784
_NB = 64
_NC = 208
_NN = 512
_KS = 16
_NOUT = _NB + _NC + _NN // _KS
_L = 16
_NCORES = 2
_NSUB = 16
_NW = _NCORES * _NSUB
_VPC = _C // _L
_BSC = 5120


def _sc_partial_rowsum(x2d, a, cvec, bs):
    rows_per_w = bs // _NW
    groups = rows_per_w // _L
    mesh = plsc.VectorSubcoreMesh(core_axis_name="c", subcore_axis_name="s")

    @functools.partial(
        pl.kernel,
        mesh=mesh,
        out_type=jax.ShapeDtypeStruct((bs * _L,), jnp.float32),
        scratch_types=[
            pltpu.VMEM((_L, _C), jnp.float32),
            pltpu.VMEM((_L, _C), jnp.float32),
            pltpu.VMEM((_L * _L,), jnp.float32),
            pltpu.VMEM((_L * _L,), jnp.float32),
            pltpu.VMEM((_C,), jnp.float32),
            pltpu.VMEM((_L,), jnp.float32),
            pltpu.SemaphoreType.DMA,
            pltpu.SemaphoreType.DMA,
            pltpu.SemaphoreType.DMA,
            pltpu.SemaphoreType.DMA,
        ],
    )
    def k(x_hbm, a_hbm, c_hbm, p_hbm, buf0, buf1, pb0, pb1, a_v, c_v,
          isem0, isem1, osem0, osem1):
        wid = lax.axis_index("s") * _NCORES + lax.axis_index("c")
        base = wid * rows_per_w
        pltpu.sync_copy(a_hbm, a_v)
        pltpu.sync_copy(c_hbm, c_v)
        cv = c_v[...]

        bufs = (buf0, buf1)
        pbs = (pb0, pb1)
        isems = (isem0, isem1)
        osems = (osem0, osem1)

        def in_slice(g):
            row0 = base + g * _L
            return x_hbm.at[pl.ds(row0, _L), :]

        def out_slice(g):
            row0 = base + g * _L
            return p_hbm.at[pl.ds(row0 * _L, _L * _L)]

        pltpu.async_copy(in_slice(0), buf0, isem0)

        def step(i, carry):
            for s in range(2):
                g = 2 * i + s
                buf, pb = bufs[s], pbs[s]
                isem, osem = isems[s], osems[s]
                o = 1 - s

                @pl.when(g + 1 < groups)
                def _():
                    pltpu.async_copy(in_slice(g + 1), bufs[o], isems[o])

                pltpu.make_async_copy(in_slice(g), buf, isem).wait()

                @pl.when(i > 0)
                def _():
                    pltpu.make_async_copy(pb, out_slice(g), osem).wait()

                for half in range(2):
                    @plsc.parallel_loop(0, _VPC, 1, unroll=7,
                                        carry=(cv,) * (_L // 2))
                    def jloop(j, accs, half=half):
                        off = j * _L
                        aj = a_v[pl.ds(off, _L)]
                        return tuple(
                            accs[r]
                            + buf[half * (_L // 2) + r, pl.ds(off, _L)] * aj
                            for r in range(_L // 2))
                    accs = jloop
                    for r in range(_L // 2):
                        rr = half * (_L // 2) + r
                        pb[pl.ds(rr * _L, _L)] = accs[r]
                pltpu.async_copy(pb, out_slice(g), osem)
            return carry

        lax.fori_loop(0, groups // 2, step, 0)
        pltpu.make_async_copy(pb0, out_slice(groups - 2), osem0).wait()
        pltpu.make_async_copy(pb1, out_slice(groups - 1), osem1).wait()

    return k(x2d, a, cvec)


def _tc_final_rowsum(p, moh, bs):
    prows = bs * _L // 128

    def body(p_ref, m_ref, o_ref):
        o_ref[...] = jax.lax.dot(
            p_ref[...], m_ref[...],
            precision=jax.lax.Precision.HIGHEST,
            preferred_element_type=jnp.float32)

    return pl.pallas_call(
        body,
        out_shape=jax.ShapeDtypeStruct((prows, 8), jnp.float32),
        grid=(2,),
        in_specs=[
            pl.BlockSpec((prows // 2, 128), lambda i: (i, 0)),
            pl.BlockSpec((128, 8), lambda i: (0, 0)),
        ],
        out_specs=pl.BlockSpec((prows // 2, 8), lambda i: (i, 0)),
    )(p, moh)


def _tc_main_rowsum(x2d, a2d, c11, row0, nrows):
    rb = 1024
    nblk = nrows // rb
    blk0 = row0 // rb

    def body(x_ref, a_ref, c_ref, o_ref):
        o_ref[...] = jnp.sum(x_ref[...] * a_ref[...], axis=1,
                             keepdims=True) + c_ref[...]

    return pl.pallas_call(
        body,
        out_shape=jax.ShapeDtypeStruct((nrows, 1), jnp.float32),
        grid=(nblk,),
        in_specs=[
            pl.BlockSpec((rb, _C), lambda i: (blk0 + i, 0)),
            pl.BlockSpec((1, _C), lambda i: (0, 0)),
            pl.BlockSpec((1, 1), lambda i: (0, 0)),
        ],
        out_specs=pl.BlockSpec((rb, 1), lambda i: (i, 0)),
    )(x2d, a2d, c11)


def kernel(input_linear, w1, b1, w2, b2, K, train_size, num_cat_variable,
           num_num_variable, num_bin_variable,
           vectorized_cate_col_name_num_list):
    x2d = input_linear.reshape(_B, _C)

    counts = jnp.asarray(vectorized_cate_col_name_num_list, dtype=jnp.int32)
    cum = jnp.cumsum(counts)
    jj = jnp.arange(_NC, dtype=jnp.int32)
    seg = jnp.sum((cum[None, :] <= jj[:, None]).astype(jnp.int32), axis=1)
    seg = jnp.minimum(seg, _NC - 1)
    gmap = jnp.concatenate([
        jnp.arange(_NB, dtype=jnp.int32),
        _NB + seg,
        _NB + _NC + jnp.arange(_NN, dtype=jnp.int32) // _KS,
    ])
    onehot = (gmap[:, None] == jnp.arange(_NOUT, dtype=jnp.int32)[None, :])
    w2g = jnp.dot(onehot.astype(jnp.float32), w2,
                  precision=jax.lax.Precision.HIGHEST)
    a = w1 * w2g
    cconst = jnp.dot(b1, w2g, precision=jax.lax.Precision.HIGHEST) + jnp.sum(b2)
    cvec = jnp.full((_L,), cconst / _L, dtype=jnp.float32)

    lane = jnp.arange(128, dtype=jnp.int32)
    moh = (lane[:, None] // _L == jnp.arange(8, dtype=jnp.int32)[None, :])
    moh = moh.astype(jnp.float32)

    p = _sc_partial_rowsum(x2d, a, cvec, _BSC)
    out_tc = _tc_main_rowsum(x2d, a.reshape(1, _C),
                             jnp.full((1, 1), cconst, jnp.float32),
                             _BSC, _B - _BSC)
    out_sc = _tc_final_rowsum(p.reshape(_BSC * _L // 128, 128), moh, _BSC)
    return jnp.concatenate([out_sc.reshape(_BSC, 1), out_tc], axis=0)


# speedup vs baseline: 1.1322x; 1.0040x over previous
import functools

import jax
import jax.numpy as jnp
from jax import lax
from jax.experimental import pallas as pl
from jax.experimental.pallas import tpu as pltpu
from jax.experimental.pallas import tpu_sc as plsc

_B = 16384
_C = 784
_NB = 64
_NC = 208
_NN = 512
_KS = 16
_NOUT = _NB + _NC + _NN // _KS
_L = 16
_NCORES = 2
_NSUB = 16
_NW = _NCORES * _NSUB
_VPC = _C // _L
_BSC = 5120


def _sc_partial_rowsum(x2d, a, cvec, bs):
    rows_per_w = bs // _NW
    groups = rows_per_w // _L
    mesh = plsc.VectorSubcoreMesh(core_axis_name="c", subcore_axis_name="s")

    @functools.partial(
        pl.kernel,
        mesh=mesh,
        out_type=jax.ShapeDtypeStruct((bs * _L,), jnp.float32),
        scratch_types=[
            pltpu.VMEM((_L, _C), jnp.float32),
            pltpu.VMEM((_L, _C), jnp.float32),
            pltpu.VMEM((_L * _L,), jnp.float32),
            pltpu.VMEM((_L * _L,), jnp.float32),
            pltpu.VMEM((_C,), jnp.float32),
            pltpu.VMEM((_L,), jnp.float32),
            pltpu.SemaphoreType.DMA,
            pltpu.SemaphoreType.DMA,
            pltpu.SemaphoreType.DMA,
            pltpu.SemaphoreType.DMA,
            pltpu.SemaphoreType.DMA,
        ],
    )
    def k(x_hbm, a_hbm, c_hbm, p_hbm, buf0, buf1, pb0, pb1, a_v, c_v,
          isem0, isem1, osem0, osem1, wsem):
        wid = lax.axis_index("s") * _NCORES + lax.axis_index("c")
        base = wid * rows_per_w
        pltpu.async_copy(a_hbm, a_v, wsem)
        pltpu.async_copy(c_hbm, c_v, wsem)

        bufs = (buf0, buf1)
        pbs = (pb0, pb1)
        isems = (isem0, isem1)
        osems = (osem0, osem1)

        def in_slice(g):
            row0 = base + g * _L
            return x_hbm.at[pl.ds(row0, _L), :]

        def out_slice(g):
            row0 = base + g * _L
            return p_hbm.at[pl.ds(row0 * _L, _L * _L)]

        pltpu.async_copy(in_slice(0), buf0, isem0)
        pltpu.make_async_copy(a_hbm, a_v, wsem).wait()
        pltpu.make_async_copy(c_hbm, c_v, wsem).wait()
        cv = c_v[...]

        def step(i, carry):
            for s in range(2):
                g = 2 * i + s
                buf, pb = bufs[s], pbs[s]
                isem, osem = isems[s], osems[s]
                o = 1 - s

                @pl.when(g + 1 < groups)
                def _():
                    pltpu.async_copy(in_slice(g + 1), bufs[o], isems[o])

                pltpu.make_async_copy(in_slice(g), buf, isem).wait()

                @pl.when(i > 0)
                def _():
                    pltpu.make_async_copy(pb, out_slice(g), osem).wait()

                for half in range(2):
                    @plsc.parallel_loop(0, _VPC, 1, unroll=7,
                                        carry=(cv,) * (_L // 2))
                    def jloop(j, accs, half=half):
                        off = j * _L
                        aj = a_v[pl.ds(off, _L)]
                        return tuple(
                            accs[r]
                            + buf[half * (_L // 2) + r, pl.ds(off, _L)] * aj
                            for r in range(_L // 2))
                    accs = jloop
                    for r in range(_L // 2):
                        rr = half * (_L // 2) + r
                        pb[pl.ds(rr * _L, _L)] = accs[r]
                pltpu.async_copy(pb, out_slice(g), osem)
            return carry

        lax.fori_loop(0, groups // 2, step, 0)
        pltpu.make_async_copy(pb0, out_slice(groups - 2), osem0).wait()
        pltpu.make_async_copy(pb1, out_slice(groups - 1), osem1).wait()

    return k(x2d, a, cvec)


def _tc_final_rowsum(p, moh, bs):
    prows = bs * _L // 128

    def body(p_ref, m_ref, o_ref):
        o_ref[...] = jax.lax.dot(
            p_ref[...], m_ref[...],
            precision=jax.lax.Precision.HIGHEST,
            preferred_element_type=jnp.float32)

    return pl.pallas_call(
        body,
        out_shape=jax.ShapeDtypeStruct((prows, 8), jnp.float32),
        grid=(2,),
        in_specs=[
            pl.BlockSpec((prows // 2, 128), lambda i: (i, 0)),
            pl.BlockSpec((128, 8), lambda i: (0, 0)),
        ],
        out_specs=pl.BlockSpec((prows // 2, 8), lambda i: (i, 0)),
    )(p, moh)


def _tc_main_rowsum(x2d, a2d, c11, row0, nrows):
    rb = 1024
    nblk = nrows // rb
    blk0 = row0 // rb

    def body(x_ref, a_ref, c_ref, o_ref):
        o_ref[...] = jnp.sum(x_ref[...] * a_ref[...], axis=1,
                             keepdims=True) + c_ref[...]

    return pl.pallas_call(
        body,
        out_shape=jax.ShapeDtypeStruct((nrows, 1), jnp.float32),
        grid=(nblk,),
        in_specs=[
            pl.BlockSpec((rb, _C), lambda i: (blk0 + i, 0)),
            pl.BlockSpec((1, _C), lambda i: (0, 0)),
            pl.BlockSpec((1, 1), lambda i: (0, 0)),
        ],
        out_specs=pl.BlockSpec((rb, 1), lambda i: (i, 0)),
    )(x2d, a2d, c11)


def kernel(input_linear, w1, b1, w2, b2, K, train_size, num_cat_variable,
           num_num_variable, num_bin_variable,
           vectorized_cate_col_name_num_list):
    x2d = input_linear.reshape(_B, _C)

    counts = jnp.asarray(vectorized_cate_col_name_num_list, dtype=jnp.int32)
    cum = jnp.cumsum(counts)
    jj = jnp.arange(_NC, dtype=jnp.int32)
    seg = jnp.sum((cum[None, :] <= jj[:, None]).astype(jnp.int32), axis=1)
    seg = jnp.minimum(seg, _NC - 1)
    gmap = jnp.concatenate([
        jnp.arange(_NB, dtype=jnp.int32),
        _NB + seg,
        _NB + _NC + jnp.arange(_NN, dtype=jnp.int32) // _KS,
    ])
    onehot = (gmap[:, None] == jnp.arange(_NOUT, dtype=jnp.int32)[None, :])
    w2g = jnp.dot(onehot.astype(jnp.float32), w2,
                  precision=jax.lax.Precision.HIGHEST)
    a = w1 * w2g
    cconst = jnp.dot(b1, w2g, precision=jax.lax.Precision.HIGHEST) + jnp.sum(b2)
    cvec = jnp.full((_L,), cconst / _L, dtype=jnp.float32)

    lane = jnp.arange(128, dtype=jnp.int32)
    moh = (lane[:, None] // _L == jnp.arange(8, dtype=jnp.int32)[None, :])
    moh = moh.astype(jnp.float32)

    p = _sc_partial_rowsum(x2d, a, cvec, _BSC)
    out_tc = _tc_main_rowsum(x2d, a.reshape(1, _C),
                             jnp.full((1, 1), cconst, jnp.float32),
                             _BSC, _B - _BSC)
    out_sc = _tc_final_rowsum(p.reshape(_BSC * _L // 128, 128), moh, _BSC)
    return jnp.concatenate([out_sc.reshape(_BSC, 1), out_tc], axis=0)
